# baseline (speedup 1.0000x reference)
import functools

import jax
import jax.numpy as jnp
from jax import lax
from jax.experimental import pallas as pl
from jax.experimental.pallas import tpu as pltpu
from jax.experimental.pallas import tpu_sc as plsc

_N = 10000
_E = 320000
_D = 128
_H = 128
_C = 64

_NC = 2
_NS = 16
_NW = _NC * _NS

_K = 128
_EROWS = 2560
_EPAD = _EROWS * _K
_RPT = _EROWS // _NW

_NACC = _N
_ZR = 632
_ZL = _NACC - 15 * _ZR

_DA = 144


def _make_agg(d, ch):
  mesh = plsc.VectorSubcoreMesh(
      core_axis_name="c", subcore_axis_name="s",
      num_cores=_NC, num_subcores=_NS)

  @functools.partial(
      pl.kernel,
      mesh=mesh,
      out_type=jax.ShapeDtypeStruct((_NC, _NACC, d), jnp.float32),
      scratch_types=[
          pltpu.VMEM((ch, _K), jnp.int32),
          pltpu.VMEM((ch, _K), jnp.int32),
          pltpu.VMEM((_K, d), jnp.float32),
          pltpu.VMEM((_K, d), jnp.float32),
          pltpu.VMEM_SHARED((_NACC, d), jnp.float32),
          pltpu.SemaphoreType.DMA,
          pltpu.SemaphoreType.DMA,
          pltpu.SemaphoreType.DMA,
          pltpu.SemaphoreType.DMA,
          pltpu.SemaphoreType.DMA,
          pltpu.SemaphoreType.DMA,
      ],
      compiler_params=pltpu.CompilerParams(use_tc_tiling_on_sc=False),
  )
  def agg(table_hbm, src_hbm, dst_hbm, zeros_hbm, out_hbm,
          src_v, dst_v, rows0, rows1, acc_sh, isem0, isem1,
          gsem0, gsem1, ssem0, ssem1):
    cid = lax.axis_index("c")
    sid = lax.axis_index("s")
    my_base = (sid * _NC + cid) * _RPT
    nch = _RPT // ch
    rows = (rows0, rows1)
    gsem = (gsem0, gsem1)
    ssem = (ssem0, ssem1)

    with jax.named_scope("agg_zero"):
      @pl.when(sid < 15)
      def _():
        pltpu.sync_copy(zeros_hbm, acc_sh.at[pl.ds(sid * _ZR, _ZR)])

      @pl.when(sid == 15)
      def _():
        pltpu.sync_copy(zeros_hbm.at[pl.ds(0, _ZL)],
                        acc_sh.at[pl.ds(15 * _ZR, _ZL)])

      plsc.subcore_barrier()

    def start_gather(j, b):
      pltpu.async_copy(table_hbm.at[src_v.at[j]], rows[b], gsem[b])

    def wait_gather(b):
      pltpu.make_async_copy(table_hbm.at[src_v.at[0]], rows[b], gsem[b]).wait()

    def start_scatter(j, b):
      pltpu.async_copy(rows[b], acc_sh.at[dst_v.at[j]], ssem[b], add=True)

    def wait_scatter(b):
      pltpu.make_async_copy(rows[b], acc_sh.at[dst_v.at[0]], ssem[b]).wait()

    def chunk(c, carry):
      base = my_base + c * ch
      a = pltpu.async_copy(src_hbm.at[pl.ds(base, ch)], src_v, isem0)
      b = pltpu.async_copy(dst_hbm.at[pl.ds(base, ch)], dst_v, isem1)
      a.wait()
      b.wait()

      start_gather(0, 0)

      def pair(jj, carry2):
        j0 = 2 * jj

        @pl.when(jj >= 1)
        def _():
          wait_scatter(1)
        start_gather(j0 + 1, 1)
        wait_gather(0)
        start_scatter(j0, 0)

        wait_scatter(0)

        @pl.when(jj + 1 < ch // 2)
        def _():
          start_gather(j0 + 2, 0)
        wait_gather(1)
        start_scatter(j0 + 1, 1)
        return carry2

      lax.fori_loop(0, ch // 2, pair, 0)
      wait_scatter(1)
      return carry

    with jax.named_scope("agg_loop"):
      lax.fori_loop(0, nch, chunk, 0)

    with jax.named_scope("agg_bar2"):
      plsc.subcore_barrier()

    with jax.named_scope("agg_out"):
      @pl.when(sid < 15)
      def _():
        pltpu.sync_copy(acc_sh.at[pl.ds(sid * _ZR, _ZR)],
                        out_hbm.at[cid, pl.ds(sid * _ZR, _ZR)])

      @pl.when(sid == 15)
      def _():
        pltpu.sync_copy(acc_sh.at[pl.ds(15 * _ZR, _ZL)],
                        out_hbm.at[cid, pl.ds(15 * _ZR, _ZL)])

  return agg


_agg_da = _make_agg(_DA, 16)
_agg_c = _make_agg(_C, _RPT)

_RB = 2000


def _tc1_body(x_ref, a0_ref, a1_ref, ws1_ref, wn1_ref, b1_ref,
              ws2_ref, wn2_ref, hs2_ref, hw2_ref, rdeg_ref):
  a0 = a0_ref[0]
  a1 = a1_ref[0]
  s = a0[:, :_D] + a1[:, :_D]
  deg = a0[:, _D:_D + 1] + a1[:, _D:_D + 1]
  rdeg = 1.0 / jnp.maximum(deg, 1.0)
  mean = s * rdeg
  x = x_ref[...]
  dot = functools.partial(
      jax.lax.dot_general,
      dimension_numbers=(((1,), (1,)), ((), ())),
      preferred_element_type=jnp.float32,
      precision=jax.lax.Precision.HIGHEST)
  h = dot(x, ws1_ref[...]) + dot(mean, wn1_ref[...]) + b1_ref[...]
  h = jnp.maximum(h, 0.0)
  hs2_ref[...] = dot(h, ws2_ref[...])
  hw2_ref[...] = dot(h, wn2_ref[...])
  rdeg_ref[...] = jnp.broadcast_to(rdeg, (_RB, 8))


def _tc1(x, acc1, ws1, wn1, b1, ws2, wn2):
  grid = _N // _RB
  return pl.pallas_call(
      _tc1_body,
      grid=(grid,),
      in_specs=[
          pl.BlockSpec((_RB, _D), lambda i: (i, 0)),
          pl.BlockSpec((1, _RB, _DA), lambda i: (0, i, 0)),
          pl.BlockSpec((1, _RB, _DA), lambda i: (1, i, 0)),
          pl.BlockSpec((_H, _D), lambda i: (0, 0)),
          pl.BlockSpec((_H, _D), lambda i: (0, 0)),
          pl.BlockSpec((1, _H), lambda i: (0, 0)),
          pl.BlockSpec((_C, _H), lambda i: (0, 0)),
          pl.BlockSpec((_C, _H), lambda i: (0, 0)),
      ],
      out_specs=[
          pl.BlockSpec((_RB, _C), lambda i: (i, 0)),
          pl.BlockSpec((_RB, _C), lambda i: (i, 0)),
          pl.BlockSpec((_RB, 8), lambda i: (i, 0)),
      ],
      out_shape=[
          jax.ShapeDtypeStruct((_N, _C), jnp.float32),
          jax.ShapeDtypeStruct((_N, _C), jnp.float32),
          jax.ShapeDtypeStruct((_N, 8), jnp.float32),
      ],
  )(x, acc1, acc1, ws1, wn1, b1, ws2, wn2)


def _tc2_body(hs2_ref, a0_ref, a1_ref, rdeg_ref, b2_ref, out_ref):
  mean = (a0_ref[0] + a1_ref[0]) * rdeg_ref[:, :1]
  out_ref[...] = hs2_ref[...] + mean + b2_ref[...]


def _tc2(hs2, acc2, rdeg, b2):
  grid = _N // _RB
  return pl.pallas_call(
      _tc2_body,
      grid=(grid,),
      in_specs=[
          pl.BlockSpec((_RB, _C), lambda i: (i, 0)),
          pl.BlockSpec((1, _RB, _C), lambda i: (0, i, 0)),
          pl.BlockSpec((1, _RB, _C), lambda i: (1, i, 0)),
          pl.BlockSpec((_RB, 8), lambda i: (i, 0)),
          pl.BlockSpec((1, _C), lambda i: (0, 0)),
      ],
      out_specs=pl.BlockSpec((_RB, _C), lambda i: (i, 0)),
      out_shape=jax.ShapeDtypeStruct((_N, _C), jnp.float32),
  )(hs2, acc2, acc2, rdeg, b2)


def kernel(x, edge_index, W_self1, W_neigh1, b1, W_self2, W_neigh2, b2):
  src = edge_index[0]
  dst = edge_index[1]
  pad = _EPAD - _E
  src_p = jnp.concatenate([src, jnp.full((pad,), _N, jnp.int32)]).reshape(_EROWS, _K)
  dst_p = jnp.concatenate([dst, jnp.arange(pad, dtype=jnp.int32)]).reshape(_EROWS, _K)

  x_aug = jnp.concatenate(
      [x, jnp.ones((_N, 1), jnp.float32), jnp.zeros((_N, _DA - _D - 1), jnp.float32)],
      axis=1)
  x_aug = jnp.concatenate([x_aug, jnp.zeros((1, _DA), jnp.float32)], axis=0)
  zeros_da = jnp.zeros((_ZR, _DA), jnp.float32)
  zeros_c = jnp.zeros((_ZR, _C), jnp.float32)

  acc1 = _agg_da(x_aug, src_p, dst_p, zeros_da)
  hs2, hw2, rdeg = _tc1(x, acc1, W_self1, W_neigh1, b1.reshape(1, _H),
                        W_self2, W_neigh2)
  hw2_pad = jnp.concatenate([hw2, jnp.zeros((1, _C), jnp.float32)], axis=0)
  acc2 = _agg_c(hw2_pad, src_p, dst_p, zeros_c)
  return _tc2(hs2, acc2, rdeg, b2.reshape(1, _C))


# speedup vs baseline: 1.0606x; 1.0606x over previous
import functools

import jax
import jax.numpy as jnp
from jax import lax
from jax.experimental import pallas as pl
from jax.experimental.pallas import tpu as pltpu
from jax.experimental.pallas import tpu_sc as plsc

_N = 10000
_E = 320000
_D = 128
_H = 128
_C = 64

_NC = 2
_NS = 16
_NW = _NC * _NS

_K = 128
_EROWS = 2560
_EPAD = _EROWS * _K
_RPT = _EROWS // _NW
_RPT_F = 120
_RPT_S = 40

_NACC = _N
_ZR = 632
_ZL = _NACC - 15 * _ZR

_DA = 144


def _make_agg(d, ch):
  mesh = plsc.VectorSubcoreMesh(
      core_axis_name="c", subcore_axis_name="s",
      num_cores=_NC, num_subcores=_NS)

  @functools.partial(
      pl.kernel,
      mesh=mesh,
      out_type=jax.ShapeDtypeStruct((_NC, _NACC, d), jnp.float32),
      scratch_types=[
          pltpu.VMEM((ch, _K), jnp.int32),
          pltpu.VMEM((ch, _K), jnp.int32),
          pltpu.VMEM((_K, d), jnp.float32),
          pltpu.VMEM((_K, d), jnp.float32),
          pltpu.VMEM_SHARED((_NACC, d), jnp.float32),
          pltpu.SemaphoreType.DMA,
          pltpu.SemaphoreType.DMA,
          pltpu.SemaphoreType.DMA,
          pltpu.SemaphoreType.DMA,
          pltpu.SemaphoreType.DMA,
          pltpu.SemaphoreType.DMA,
      ],
      compiler_params=pltpu.CompilerParams(use_tc_tiling_on_sc=False),
  )
  def agg(table_hbm, src_hbm, dst_hbm, zeros_hbm, out_hbm,
          src_v, dst_v, rows0, rows1, acc_sh, isem0, isem1,
          gsem0, gsem1, ssem0, ssem1):
    cid = lax.axis_index("c")
    sid = lax.axis_index("s")
    my_base = jnp.where(cid == 0, sid * _RPT_F, 16 * _RPT_F + sid * _RPT_S)
    nch = jnp.where(cid == 0, _RPT_F // ch, _RPT_S // ch)
    rows = (rows0, rows1)
    gsem = (gsem0, gsem1)
    ssem = (ssem0, ssem1)

    with jax.named_scope("agg_zero"):
      @pl.when(sid < 15)
      def _():
        pltpu.sync_copy(zeros_hbm, acc_sh.at[pl.ds(sid * _ZR, _ZR)])

      @pl.when(sid == 15)
      def _():
        pltpu.sync_copy(zeros_hbm.at[pl.ds(0, _ZL)],
                        acc_sh.at[pl.ds(15 * _ZR, _ZL)])

      plsc.subcore_barrier()

    def start_gather(j, b):
      pltpu.async_copy(table_hbm.at[src_v.at[j]], rows[b], gsem[b])

    def wait_gather(b):
      pltpu.make_async_copy(table_hbm.at[src_v.at[0]], rows[b], gsem[b]).wait()

    def start_scatter(j, b):
      pltpu.async_copy(rows[b], acc_sh.at[dst_v.at[j]], ssem[b], add=True)

    def wait_scatter(b):
      pltpu.make_async_copy(rows[b], acc_sh.at[dst_v.at[0]], ssem[b]).wait()

    def chunk(c, carry):
      base = my_base + c * ch
      a = pltpu.async_copy(src_hbm.at[pl.ds(base, ch)], src_v, isem0)
      b = pltpu.async_copy(dst_hbm.at[pl.ds(base, ch)], dst_v, isem1)
      a.wait()
      b.wait()

      start_gather(0, 0)

      def pair(jj, carry2):
        j0 = 2 * jj

        @pl.when(jj >= 1)
        def _():
          wait_scatter(1)
        start_gather(j0 + 1, 1)
        wait_gather(0)
        start_scatter(j0, 0)

        wait_scatter(0)

        @pl.when(jj + 1 < ch // 2)
        def _():
          start_gather(j0 + 2, 0)
        wait_gather(1)
        start_scatter(j0 + 1, 1)
        return carry2

      lax.fori_loop(0, ch // 2, pair, 0)
      wait_scatter(1)
      return carry

    with jax.named_scope("agg_loop"):
      lax.fori_loop(0, nch, chunk, 0)

    with jax.named_scope("agg_bar2"):
      plsc.subcore_barrier()

    with jax.named_scope("agg_out"):
      @pl.when(sid < 15)
      def _():
        pltpu.sync_copy(acc_sh.at[pl.ds(sid * _ZR, _ZR)],
                        out_hbm.at[cid, pl.ds(sid * _ZR, _ZR)])

      @pl.when(sid == 15)
      def _():
        pltpu.sync_copy(acc_sh.at[pl.ds(15 * _ZR, _ZL)],
                        out_hbm.at[cid, pl.ds(15 * _ZR, _ZL)])

  return agg


_agg_da = _make_agg(_DA, 8)
_agg_c = _make_agg(_C, 40)

_RB = 2000


def _tc1_body(x_ref, a0_ref, a1_ref, ws1_ref, wn1_ref, b1_ref,
              ws2_ref, wn2_ref, hs2_ref, hw2_ref, rdeg_ref):
  a0 = a0_ref[0]
  a1 = a1_ref[0]
  s = a0[:, :_D] + a1[:, :_D]
  deg = a0[:, _D:_D + 1] + a1[:, _D:_D + 1]
  rdeg = 1.0 / jnp.maximum(deg, 1.0)
  mean = s * rdeg
  x = x_ref[...]
  dot = functools.partial(
      jax.lax.dot_general,
      dimension_numbers=(((1,), (1,)), ((), ())),
      preferred_element_type=jnp.float32,
      precision=jax.lax.Precision.HIGHEST)
  h = dot(x, ws1_ref[...]) + dot(mean, wn1_ref[...]) + b1_ref[...]
  h = jnp.maximum(h, 0.0)
  hs2_ref[...] = dot(h, ws2_ref[...])
  hw2_ref[...] = dot(h, wn2_ref[...])
  rdeg_ref[...] = jnp.broadcast_to(rdeg, (_RB, 8))


def _tc1(x, acc1, ws1, wn1, b1, ws2, wn2):
  grid = _N // _RB
  return pl.pallas_call(
      _tc1_body,
      grid=(grid,),
      in_specs=[
          pl.BlockSpec((_RB, _D), lambda i: (i, 0)),
          pl.BlockSpec((1, _RB, _DA), lambda i: (0, i, 0)),
          pl.BlockSpec((1, _RB, _DA), lambda i: (1, i, 0)),
          pl.BlockSpec((_H, _D), lambda i: (0, 0)),
          pl.BlockSpec((_H, _D), lambda i: (0, 0)),
          pl.BlockSpec((1, _H), lambda i: (0, 0)),
          pl.BlockSpec((_C, _H), lambda i: (0, 0)),
          pl.BlockSpec((_C, _H), lambda i: (0, 0)),
      ],
      out_specs=[
          pl.BlockSpec((_RB, _C), lambda i: (i, 0)),
          pl.BlockSpec((_RB, _C), lambda i: (i, 0)),
          pl.BlockSpec((_RB, 8), lambda i: (i, 0)),
      ],
      out_shape=[
          jax.ShapeDtypeStruct((_N, _C), jnp.float32),
          jax.ShapeDtypeStruct((_N, _C), jnp.float32),
          jax.ShapeDtypeStruct((_N, 8), jnp.float32),
      ],
  )(x, acc1, acc1, ws1, wn1, b1, ws2, wn2)


def _tc2_body(hs2_ref, a0_ref, a1_ref, rdeg_ref, b2_ref, out_ref):
  mean = (a0_ref[0] + a1_ref[0]) * rdeg_ref[:, :1]
  out_ref[...] = hs2_ref[...] + mean + b2_ref[...]


def _tc2(hs2, acc2, rdeg, b2):
  grid = _N // _RB
  return pl.pallas_call(
      _tc2_body,
      grid=(grid,),
      in_specs=[
          pl.BlockSpec((_RB, _C), lambda i: (i, 0)),
          pl.BlockSpec((1, _RB, _C), lambda i: (0, i, 0)),
          pl.BlockSpec((1, _RB, _C), lambda i: (1, i, 0)),
          pl.BlockSpec((_RB, 8), lambda i: (i, 0)),
          pl.BlockSpec((1, _C), lambda i: (0, 0)),
      ],
      out_specs=pl.BlockSpec((_RB, _C), lambda i: (i, 0)),
      out_shape=jax.ShapeDtypeStruct((_N, _C), jnp.float32),
  )(hs2, acc2, acc2, rdeg, b2)


def kernel(x, edge_index, W_self1, W_neigh1, b1, W_self2, W_neigh2, b2):
  src = edge_index[0]
  dst = edge_index[1]
  pad = _EPAD - _E
  src_p = jnp.concatenate([src, jnp.full((pad,), _N, jnp.int32)]).reshape(_EROWS, _K)
  dst_p = jnp.concatenate([dst, jnp.arange(pad, dtype=jnp.int32)]).reshape(_EROWS, _K)

  x_aug = jnp.concatenate(
      [x, jnp.ones((_N, 1), jnp.float32), jnp.zeros((_N, _DA - _D - 1), jnp.float32)],
      axis=1)
  x_aug = jnp.concatenate([x_aug, jnp.zeros((1, _DA), jnp.float32)], axis=0)
  zeros_da = jnp.zeros((_ZR, _DA), jnp.float32)
  zeros_c = jnp.zeros((_ZR, _C), jnp.float32)

  acc1 = _agg_da(x_aug, src_p, dst_p, zeros_da)
  hs2, hw2, rdeg = _tc1(x, acc1, W_self1, W_neigh1, b1.reshape(1, _H),
                        W_self2, W_neigh2)
  hw2_pad = jnp.concatenate([hw2, jnp.zeros((1, _C), jnp.float32)], axis=0)
  acc2 = _agg_c(hw2_pad, src_p, dst_p, zeros_c)
  return _tc2(hs2, acc2, rdeg, b2.reshape(1, _C))


# speedup vs baseline: 1.9123x; 1.8030x over previous
import functools

import jax
import jax.numpy as jnp
from jax import lax
from jax.experimental import pallas as pl
from jax.experimental.pallas import tpu as pltpu
from jax.experimental.pallas import tpu_sc as plsc

_N = 10000
_E = 320000
_D = 128
_H = 128
_C = 64

_NC = 2
_NS = 16
_NW = _NC * _NS

_K = 125
_EROWS = 2560
_RPT = _EROWS // _NW
_RPT_F = 120
_RPT_S = 40

_NACC = _N
_ZR = 632
_ZL = _NACC - 15 * _ZR

_DA = 144


def _make_agg(d, ch):
  mesh = plsc.VectorSubcoreMesh(
      core_axis_name="c", subcore_axis_name="s",
      num_cores=_NC, num_subcores=_NS)

  @functools.partial(
      pl.kernel,
      mesh=mesh,
      out_type=jax.ShapeDtypeStruct((_NC, _NACC, d), jnp.float32),
      scratch_types=[
          pltpu.VMEM((ch, _K), jnp.int32),
          pltpu.VMEM((ch, _K), jnp.int32),
          pltpu.VMEM((_K, d), jnp.float32),
          pltpu.VMEM((_K, d), jnp.float32),
          pltpu.VMEM_SHARED((_NACC, d), jnp.float32),
          pltpu.SemaphoreType.DMA,
          pltpu.SemaphoreType.DMA,
          pltpu.SemaphoreType.DMA,
          pltpu.SemaphoreType.DMA,
          pltpu.SemaphoreType.DMA,
          pltpu.SemaphoreType.DMA,
      ],
      compiler_params=pltpu.CompilerParams(use_tc_tiling_on_sc=False),
  )
  def agg(table_hbm, src_hbm, dst_hbm, zeros_hbm, out_hbm,
          src_v, dst_v, rows0, rows1, acc_sh, isem0, isem1,
          gsem0, gsem1, ssem0, ssem1):
    cid = lax.axis_index("c")
    sid = lax.axis_index("s")
    my_base = jnp.where(cid == 0, sid * _RPT_F, 16 * _RPT_F + sid * _RPT_S)
    nch = jnp.where(cid == 0, _RPT_F // ch, _RPT_S // ch)
    rows = (rows0, rows1)
    gsem = (gsem0, gsem1)
    ssem = (ssem0, ssem1)

    with jax.named_scope("agg_zero"):
      @pl.when(sid < 15)
      def _():
        pltpu.sync_copy(zeros_hbm, acc_sh.at[pl.ds(sid * _ZR, _ZR)])

      @pl.when(sid == 15)
      def _():
        pltpu.sync_copy(zeros_hbm.at[pl.ds(0, _ZL)],
                        acc_sh.at[pl.ds(15 * _ZR, _ZL)])

      plsc.subcore_barrier()

    def start_gather(j, b):
      pltpu.async_copy(table_hbm.at[src_v.at[j]], rows[b], gsem[b])

    def wait_gather(b):
      pltpu.make_async_copy(table_hbm.at[src_v.at[0]], rows[b], gsem[b]).wait()

    def start_scatter(j, b):
      pltpu.async_copy(rows[b], acc_sh.at[dst_v.at[j]], ssem[b], add=True)

    def wait_scatter(b):
      pltpu.make_async_copy(rows[b], acc_sh.at[dst_v.at[0]], ssem[b]).wait()

    def chunk(c, carry):
      base = my_base + c * ch
      a = pltpu.async_copy(src_hbm.at[pl.ds(base, ch)], src_v, isem0)
      b = pltpu.async_copy(dst_hbm.at[pl.ds(base, ch)], dst_v, isem1)
      a.wait()
      b.wait()

      start_gather(0, 0)

      def pair(jj, carry2):
        j0 = 2 * jj

        @pl.when(jj >= 1)
        def _():
          wait_scatter(1)
        start_gather(j0 + 1, 1)
        wait_gather(0)
        start_scatter(j0, 0)

        wait_scatter(0)

        @pl.when(jj + 1 < ch // 2)
        def _():
          start_gather(j0 + 2, 0)
        wait_gather(1)
        start_scatter(j0 + 1, 1)
        return carry2

      lax.fori_loop(0, ch // 2, pair, 0)
      wait_scatter(1)
      return carry

    with jax.named_scope("agg_loop"):
      lax.fori_loop(0, nch, chunk, 0)

    with jax.named_scope("agg_bar2"):
      plsc.subcore_barrier()

    with jax.named_scope("agg_out"):
      @pl.when(sid < 15)
      def _():
        pltpu.sync_copy(acc_sh.at[pl.ds(sid * _ZR, _ZR)],
                        out_hbm.at[cid, pl.ds(sid * _ZR, _ZR)])

      @pl.when(sid == 15)
      def _():
        pltpu.sync_copy(acc_sh.at[pl.ds(15 * _ZR, _ZL)],
                        out_hbm.at[cid, pl.ds(15 * _ZR, _ZL)])

  return agg


_agg_da = _make_agg(_DA, 8)
_agg_c = _make_agg(_C, 40)

_RB = 2000


def _tc1_body(x_ref, a0_ref, a1_ref, ws1_ref, wn1_ref, b1_ref,
              ws2_ref, wn2_ref, hs2_ref, hw2_ref, rdeg_ref):
  a0 = a0_ref[0]
  a1 = a1_ref[0]
  s = a0[:, :_D] + a1[:, :_D]
  deg = a0[:, _D:_D + 1] + a1[:, _D:_D + 1]
  rdeg = 1.0 / jnp.maximum(deg, 1.0)
  mean = s * rdeg
  x = x_ref[...]
  dot = functools.partial(
      jax.lax.dot_general,
      dimension_numbers=(((1,), (1,)), ((), ())),
      preferred_element_type=jnp.float32,
      precision=jax.lax.Precision.HIGHEST)
  h = dot(x, ws1_ref[...]) + dot(mean, wn1_ref[...]) + b1_ref[...]
  h = jnp.maximum(h, 0.0)
  hs2_ref[...] = dot(h, ws2_ref[...])
  hw2_ref[...] = dot(h, wn2_ref[...])
  rdeg_ref[...] = jnp.broadcast_to(rdeg, (_RB, 8))


def _tc1(x, acc1, ws1, wn1, b1, ws2, wn2):
  grid = _N // _RB
  return pl.pallas_call(
      _tc1_body,
      grid=(grid,),
      in_specs=[
          pl.BlockSpec((_RB, _D), lambda i: (i, 0)),
          pl.BlockSpec((1, _RB, _DA), lambda i: (0, i, 0)),
          pl.BlockSpec((1, _RB, _DA), lambda i: (1, i, 0)),
          pl.BlockSpec((_H, _D), lambda i: (0, 0)),
          pl.BlockSpec((_H, _D), lambda i: (0, 0)),
          pl.BlockSpec((1, _H), lambda i: (0, 0)),
          pl.BlockSpec((_C, _H), lambda i: (0, 0)),
          pl.BlockSpec((_C, _H), lambda i: (0, 0)),
      ],
      out_specs=[
          pl.BlockSpec((_RB, _C), lambda i: (i, 0)),
          pl.BlockSpec((_RB, _C), lambda i: (i, 0)),
          pl.BlockSpec((_RB, 8), lambda i: (i, 0)),
      ],
      out_shape=[
          jax.ShapeDtypeStruct((_N, _C), jnp.float32),
          jax.ShapeDtypeStruct((_N, _C), jnp.float32),
          jax.ShapeDtypeStruct((_N, 8), jnp.float32),
      ],
  )(x, acc1, acc1, ws1, wn1, b1, ws2, wn2)


def _tc2_body(hs2_ref, a0_ref, a1_ref, rdeg_ref, b2_ref, out_ref):
  mean = (a0_ref[0] + a1_ref[0]) * rdeg_ref[:, :1]
  out_ref[...] = hs2_ref[...] + mean + b2_ref[...]


def _tc2(hs2, acc2, rdeg, b2):
  grid = _N // _RB
  return pl.pallas_call(
      _tc2_body,
      grid=(grid,),
      in_specs=[
          pl.BlockSpec((_RB, _C), lambda i: (i, 0)),
          pl.BlockSpec((1, _RB, _C), lambda i: (0, i, 0)),
          pl.BlockSpec((1, _RB, _C), lambda i: (1, i, 0)),
          pl.BlockSpec((_RB, 8), lambda i: (i, 0)),
          pl.BlockSpec((1, _C), lambda i: (0, 0)),
      ],
      out_specs=pl.BlockSpec((_RB, _C), lambda i: (i, 0)),
      out_shape=jax.ShapeDtypeStruct((_N, _C), jnp.float32),
  )(hs2, acc2, acc2, rdeg, b2)


def kernel(x, edge_index, W_self1, W_neigh1, b1, W_self2, W_neigh2, b2):
  src_p = edge_index[0].reshape(_EROWS, _K)
  dst_p = edge_index[1].reshape(_EROWS, _K)

  x_aug = jnp.concatenate(
      [x, jnp.ones((_N, 1), jnp.float32), jnp.zeros((_N, _DA - _D - 1), jnp.float32)],
      axis=1)
  zeros_da = jnp.zeros((_ZR, _DA), jnp.float32)
  zeros_c = jnp.zeros((_ZR, _C), jnp.float32)

  acc1 = _agg_da(x_aug, src_p, dst_p, zeros_da)
  hs2, hw2, rdeg = _tc1(x, acc1, W_self1, W_neigh1, b1.reshape(1, _H),
                        W_self2, W_neigh2)
  acc2 = _agg_c(hw2, src_p, dst_p, zeros_c)
  return _tc2(hs2, acc2, rdeg, b2.reshape(1, _C))


# speedup vs baseline: 2.4465x; 1.2794x over previous
import functools

import jax
import jax.numpy as jnp
from jax import lax
from jax.experimental import pallas as pl
from jax.experimental.pallas import tpu as pltpu
from jax.experimental.pallas import tpu_sc as plsc

_N = 10000
_E = 320000
_D = 128
_H = 128
_C = 64

_NC = 2
_NS = 16
_NW = _NC * _NS

_K = 125
_EROWS = 2560
_RPT = _EROWS // _NW
_RPT_F = 80
_RPT_S = 80

_NACC = _N
_ZR = 632
_ZL = _NACC - 15 * _ZR

_DA = 144


def _make_agg(d, ch):
  mesh = plsc.VectorSubcoreMesh(
      core_axis_name="c", subcore_axis_name="s",
      num_cores=_NC, num_subcores=_NS)

  @functools.partial(
      pl.kernel,
      mesh=mesh,
      out_type=jax.ShapeDtypeStruct((_NC, _NACC, d), jnp.float32),
      scratch_types=[
          pltpu.VMEM((ch, _K), jnp.int32),
          pltpu.VMEM((ch, _K), jnp.int32),
          pltpu.VMEM((_K, d), jnp.float32),
          pltpu.VMEM((_K, d), jnp.float32),
          pltpu.VMEM_SHARED((_NACC, d), jnp.float32),
          pltpu.SemaphoreType.DMA,
          pltpu.SemaphoreType.DMA,
          pltpu.SemaphoreType.DMA,
          pltpu.SemaphoreType.DMA,
          pltpu.SemaphoreType.DMA,
          pltpu.SemaphoreType.DMA,
      ],
      compiler_params=pltpu.CompilerParams(use_tc_tiling_on_sc=False),
  )
  def agg(table_hbm, src_hbm, dst_hbm, zeros_hbm, out_hbm,
          src_v, dst_v, rows0, rows1, acc_sh, isem0, isem1,
          gsem0, gsem1, ssem0, ssem1):
    cid = lax.axis_index("c")
    sid = lax.axis_index("s")
    my_base = jnp.where(cid == 0, sid * _RPT_F, 16 * _RPT_F + sid * _RPT_S)
    nch = jnp.where(cid == 0, _RPT_F // ch, _RPT_S // ch)
    rows = (rows0, rows1)
    gsem = (gsem0, gsem1)
    ssem = (ssem0, ssem1)

    with jax.named_scope("agg_zero"):
      @pl.when(sid < 15)
      def _():
        pltpu.sync_copy(zeros_hbm, acc_sh.at[pl.ds(sid * _ZR, _ZR)])

      @pl.when(sid == 15)
      def _():
        pltpu.sync_copy(zeros_hbm.at[pl.ds(0, _ZL)],
                        acc_sh.at[pl.ds(15 * _ZR, _ZL)])

      plsc.subcore_barrier()

    def start_gather(j, b):
      pltpu.async_copy(table_hbm.at[src_v.at[j]], rows[b], gsem[b])

    def wait_gather(b):
      pltpu.make_async_copy(table_hbm.at[src_v.at[0]], rows[b], gsem[b]).wait()

    def start_scatter(j, b):
      pltpu.async_copy(rows[b], acc_sh.at[dst_v.at[j]], ssem[b], add=True)

    def wait_scatter(b):
      pltpu.make_async_copy(rows[b], acc_sh.at[dst_v.at[0]], ssem[b]).wait()

    def chunk(c, carry):
      base = my_base + c * ch
      a = pltpu.async_copy(src_hbm.at[pl.ds(base, ch)], src_v, isem0)
      b = pltpu.async_copy(dst_hbm.at[pl.ds(base, ch)], dst_v, isem1)
      a.wait()
      b.wait()

      start_gather(0, 0)

      def pair(jj, carry2):
        j0 = 2 * jj

        @pl.when(jj >= 1)
        def _():
          wait_scatter(1)
        start_gather(j0 + 1, 1)
        wait_gather(0)
        start_scatter(j0, 0)

        wait_scatter(0)

        @pl.when(jj + 1 < ch // 2)
        def _():
          start_gather(j0 + 2, 0)
        wait_gather(1)
        start_scatter(j0 + 1, 1)
        return carry2

      lax.fori_loop(0, ch // 2, pair, 0)
      wait_scatter(1)
      return carry

    with jax.named_scope("agg_loop"):
      lax.fori_loop(0, nch, chunk, 0)

    with jax.named_scope("agg_bar2"):
      plsc.subcore_barrier()

    with jax.named_scope("agg_out"):
      @pl.when(sid < 15)
      def _():
        pltpu.sync_copy(acc_sh.at[pl.ds(sid * _ZR, _ZR)],
                        out_hbm.at[cid, pl.ds(sid * _ZR, _ZR)])

      @pl.when(sid == 15)
      def _():
        pltpu.sync_copy(acc_sh.at[pl.ds(15 * _ZR, _ZL)],
                        out_hbm.at[cid, pl.ds(15 * _ZR, _ZL)])

  return agg


_agg_da = _make_agg(_DA, 16)
_agg_c = _make_agg(_C, 80)

_RB = 2000


def _tc1_body(x_ref, a0_ref, a1_ref, ws1_ref, wn1_ref, b1_ref,
              ws2_ref, wn2_ref, hs2_ref, hw2_ref, rdeg_ref):
  a0 = a0_ref[0]
  a1 = a1_ref[0]
  s = a0[:, :_D] + a1[:, :_D]
  deg = a0[:, _D:_D + 1] + a1[:, _D:_D + 1]
  rdeg = 1.0 / jnp.maximum(deg, 1.0)
  mean = s * rdeg
  x = x_ref[...]
  dot = functools.partial(
      jax.lax.dot_general,
      dimension_numbers=(((1,), (1,)), ((), ())),
      preferred_element_type=jnp.float32,
      precision=jax.lax.Precision.HIGHEST)
  h = dot(x, ws1_ref[...]) + dot(mean, wn1_ref[...]) + b1_ref[...]
  h = jnp.maximum(h, 0.0)
  hs2_ref[...] = dot(h, ws2_ref[...])
  hw2_ref[...] = dot(h, wn2_ref[...])
  rdeg_ref[...] = jnp.broadcast_to(rdeg, (_RB, 8))


def _tc1(x, acc1, ws1, wn1, b1, ws2, wn2):
  grid = _N // _RB
  return pl.pallas_call(
      _tc1_body,
      grid=(grid,),
      in_specs=[
          pl.BlockSpec((_RB, _D), lambda i: (i, 0)),
          pl.BlockSpec((1, _RB, _DA), lambda i: (0, i, 0)),
          pl.BlockSpec((1, _RB, _DA), lambda i: (1, i, 0)),
          pl.BlockSpec((_H, _D), lambda i: (0, 0)),
          pl.BlockSpec((_H, _D), lambda i: (0, 0)),
          pl.BlockSpec((1, _H), lambda i: (0, 0)),
          pl.BlockSpec((_C, _H), lambda i: (0, 0)),
          pl.BlockSpec((_C, _H), lambda i: (0, 0)),
      ],
      out_specs=[
          pl.BlockSpec((_RB, _C), lambda i: (i, 0)),
          pl.BlockSpec((_RB, _C), lambda i: (i, 0)),
          pl.BlockSpec((_RB, 8), lambda i: (i, 0)),
      ],
      out_shape=[
          jax.ShapeDtypeStruct((_N, _C), jnp.float32),
          jax.ShapeDtypeStruct((_N, _C), jnp.float32),
          jax.ShapeDtypeStruct((_N, 8), jnp.float32),
      ],
  )(x, acc1, acc1, ws1, wn1, b1, ws2, wn2)


def _tc2_body(hs2_ref, a0_ref, a1_ref, rdeg_ref, b2_ref, out_ref):
  mean = (a0_ref[0] + a1_ref[0]) * rdeg_ref[:, :1]
  out_ref[...] = hs2_ref[...] + mean + b2_ref[...]


def _tc2(hs2, acc2, rdeg, b2):
  grid = _N // _RB
  return pl.pallas_call(
      _tc2_body,
      grid=(grid,),
      in_specs=[
          pl.BlockSpec((_RB, _C), lambda i: (i, 0)),
          pl.BlockSpec((1, _RB, _C), lambda i: (0, i, 0)),
          pl.BlockSpec((1, _RB, _C), lambda i: (1, i, 0)),
          pl.BlockSpec((_RB, 8), lambda i: (i, 0)),
          pl.BlockSpec((1, _C), lambda i: (0, 0)),
      ],
      out_specs=pl.BlockSpec((_RB, _C), lambda i: (i, 0)),
      out_shape=jax.ShapeDtypeStruct((_N, _C), jnp.float32),
  )(hs2, acc2, acc2, rdeg, b2)


def kernel(x, edge_index, W_self1, W_neigh1, b1, W_self2, W_neigh2, b2):
  src_p = edge_index[0].reshape(_EROWS, _K)
  dst_p = edge_index[1].reshape(_EROWS, _K)

  x_aug = jnp.concatenate(
      [x, jnp.ones((_N, 1), jnp.float32), jnp.zeros((_N, _DA - _D - 1), jnp.float32)],
      axis=1)
  zeros_da = jnp.zeros((_ZR, _DA), jnp.float32)
  zeros_c = jnp.zeros((_ZR, _C), jnp.float32)

  acc1 = _agg_da(x_aug, src_p, dst_p, zeros_da)
  hs2, hw2, rdeg = _tc1(x, acc1, W_self1, W_neigh1, b1.reshape(1, _H),
                        W_self2, W_neigh2)
  acc2 = _agg_c(hw2, src_p, dst_p, zeros_c)
  return _tc2(hs2, acc2, rdeg, b2.reshape(1, _C))


# speedup vs baseline: 2.5973x; 1.0616x over previous
import functools

import jax
import jax.numpy as jnp
from jax import lax
from jax.experimental import pallas as pl
from jax.experimental.pallas import tpu as pltpu
from jax.experimental.pallas import tpu_sc as plsc

_N = 10000
_E = 320000
_D = 128
_H = 128
_C = 64

_NC = 2
_NS = 16
_NW = _NC * _NS

_K = 125
_EROWS = 2560
_RPT = _EROWS // _NW
_RPT_F = 80
_RPT_S = 80

_NACC = _N
_ZR = 632
_ZL = _NACC - 15 * _ZR

_DA = 144


def _make_agg(d, ch):
  mesh = plsc.VectorSubcoreMesh(
      core_axis_name="c", subcore_axis_name="s",
      num_cores=_NC, num_subcores=_NS)

  @functools.partial(
      pl.kernel,
      mesh=mesh,
      out_type=jax.ShapeDtypeStruct((_NC, _NACC, d), jnp.float32),
      scratch_types=[
          pltpu.VMEM((ch, _K), jnp.int32),
          pltpu.VMEM((ch, _K), jnp.int32),
          pltpu.VMEM((_K, d), jnp.float32),
          pltpu.VMEM((_K, d), jnp.float32),
          pltpu.VMEM_SHARED((_NACC, d), jnp.float32),
          pltpu.SemaphoreType.DMA,
          pltpu.SemaphoreType.DMA,
          pltpu.SemaphoreType.DMA,
          pltpu.SemaphoreType.DMA,
          pltpu.SemaphoreType.DMA,
          pltpu.SemaphoreType.DMA,
      ],
      compiler_params=pltpu.CompilerParams(use_tc_tiling_on_sc=False),
  )
  def agg(table_hbm, src_hbm, dst_hbm, zeros_hbm, out_hbm,
          src_v, dst_v, rows0, rows1, acc_sh, isem0, isem1,
          gsem0, gsem1, ssem0, ssem1):
    cid = lax.axis_index("c")
    sid = lax.axis_index("s")
    my_base = jnp.where(cid == 0, sid * _RPT_F, 16 * _RPT_F + sid * _RPT_S)
    nch = jnp.where(cid == 0, _RPT_F // ch, _RPT_S // ch)
    rows = (rows0, rows1)
    gsem = (gsem0, gsem1)
    ssem = (ssem0, ssem1)

    @pl.when(sid < 15)
    def _():
      pltpu.sync_copy(zeros_hbm, acc_sh.at[pl.ds(sid * _ZR, _ZR)])

    @pl.when(sid == 15)
    def _():
      pltpu.sync_copy(zeros_hbm.at[pl.ds(0, _ZL)],
                      acc_sh.at[pl.ds(15 * _ZR, _ZL)])

    plsc.subcore_barrier()

    def start_gather(j, b):
      pltpu.async_copy(table_hbm.at[src_v.at[j]], rows[b], gsem[b])

    def wait_gather(b):
      pltpu.make_async_copy(table_hbm.at[src_v.at[0]], rows[b], gsem[b]).wait()

    def start_scatter(j, b):
      pltpu.async_copy(rows[b], acc_sh.at[dst_v.at[j]], ssem[b], add=True)

    def wait_scatter(b):
      pltpu.make_async_copy(rows[b], acc_sh.at[dst_v.at[0]], ssem[b]).wait()

    def chunk(c, carry):
      base = my_base + c * ch
      a = pltpu.async_copy(src_hbm.at[pl.ds(base, ch)], src_v, isem0)
      b = pltpu.async_copy(dst_hbm.at[pl.ds(base, ch)], dst_v, isem1)
      a.wait()
      b.wait()

      start_gather(0, 0)

      def pair(jj, carry2):
        j0 = 2 * jj

        @pl.when(jj >= 1)
        def _():
          wait_scatter(1)
        start_gather(j0 + 1, 1)
        wait_gather(0)
        start_scatter(j0, 0)

        wait_scatter(0)

        @pl.when(jj + 1 < ch // 2)
        def _():
          start_gather(j0 + 2, 0)
        wait_gather(1)
        start_scatter(j0 + 1, 1)
        return carry2

      lax.fori_loop(0, ch // 2, pair, 0)
      wait_scatter(1)
      return carry

    lax.fori_loop(0, nch, chunk, 0)
    plsc.subcore_barrier()

    @pl.when(sid < 15)
    def _():
      pltpu.sync_copy(acc_sh.at[pl.ds(sid * _ZR, _ZR)],
                      out_hbm.at[cid, pl.ds(sid * _ZR, _ZR)])

    @pl.when(sid == 15)
    def _():
      pltpu.sync_copy(acc_sh.at[pl.ds(15 * _ZR, _ZL)],
                      out_hbm.at[cid, pl.ds(15 * _ZR, _ZL)])

  return agg


_agg_da = _make_agg(_DA, 16)
_agg_c = _make_agg(_C, 80)

_RB = 2000


def _tc1_body(x_ref, a0_ref, a1_ref, ws1_ref, wn1_ref, b1_ref,
              ws2_ref, wn2_ref, hs2_ref, hw2_ref, rdeg_ref):
  a0 = a0_ref[0]
  a1 = a1_ref[0]
  s = a0[:, :_D] + a1[:, :_D]
  deg = a0[:, _D:_D + 1] + a1[:, _D:_D + 1]
  rdeg = 1.0 / jnp.maximum(deg, 1.0)
  mean = s * rdeg
  x = x_ref[...]
  dot = functools.partial(
      jax.lax.dot_general,
      dimension_numbers=(((1,), (1,)), ((), ())),
      preferred_element_type=jnp.float32)
  h = dot(x, ws1_ref[...]) + dot(mean, wn1_ref[...]) + b1_ref[...]
  h = jnp.maximum(h, 0.0)
  hs2_ref[...] = dot(h, ws2_ref[...])
  hw2_ref[...] = dot(h, wn2_ref[...])
  rdeg_ref[...] = jnp.broadcast_to(rdeg, (_RB, 8))


def _tc1(x, acc1, ws1, wn1, b1, ws2, wn2):
  grid = _N // _RB
  return pl.pallas_call(
      _tc1_body,
      grid=(grid,),
      in_specs=[
          pl.BlockSpec((_RB, _D), lambda i: (i, 0)),
          pl.BlockSpec((1, _RB, _DA), lambda i: (0, i, 0)),
          pl.BlockSpec((1, _RB, _DA), lambda i: (1, i, 0)),
          pl.BlockSpec((_H, _D), lambda i: (0, 0)),
          pl.BlockSpec((_H, _D), lambda i: (0, 0)),
          pl.BlockSpec((1, _H), lambda i: (0, 0)),
          pl.BlockSpec((_C, _H), lambda i: (0, 0)),
          pl.BlockSpec((_C, _H), lambda i: (0, 0)),
      ],
      out_specs=[
          pl.BlockSpec((_RB, _C), lambda i: (i, 0)),
          pl.BlockSpec((_RB, _C), lambda i: (i, 0)),
          pl.BlockSpec((_RB, 8), lambda i: (i, 0)),
      ],
      out_shape=[
          jax.ShapeDtypeStruct((_N, _C), jnp.float32),
          jax.ShapeDtypeStruct((_N, _C), jnp.float32),
          jax.ShapeDtypeStruct((_N, 8), jnp.float32),
      ],
  )(x, acc1, acc1, ws1, wn1, b1, ws2, wn2)


def _tc2_body(hs2_ref, a0_ref, a1_ref, rdeg_ref, b2_ref, out_ref):
  mean = (a0_ref[0] + a1_ref[0]) * rdeg_ref[:, :1]
  out_ref[...] = hs2_ref[...] + mean + b2_ref[...]


def _tc2(hs2, acc2, rdeg, b2):
  grid = _N // _RB
  return pl.pallas_call(
      _tc2_body,
      grid=(grid,),
      in_specs=[
          pl.BlockSpec((_RB, _C), lambda i: (i, 0)),
          pl.BlockSpec((1, _RB, _C), lambda i: (0, i, 0)),
          pl.BlockSpec((1, _RB, _C), lambda i: (1, i, 0)),
          pl.BlockSpec((_RB, 8), lambda i: (i, 0)),
          pl.BlockSpec((1, _C), lambda i: (0, 0)),
      ],
      out_specs=pl.BlockSpec((_RB, _C), lambda i: (i, 0)),
      out_shape=jax.ShapeDtypeStruct((_N, _C), jnp.float32),
  )(hs2, acc2, acc2, rdeg, b2)


def kernel(x, edge_index, W_self1, W_neigh1, b1, W_self2, W_neigh2, b2):
  src_p = edge_index[0].reshape(_EROWS, _K)
  dst_p = edge_index[1].reshape(_EROWS, _K)

  x_aug = jnp.concatenate(
      [x, jnp.ones((_N, 1), jnp.float32), jnp.zeros((_N, _DA - _D - 1), jnp.float32)],
      axis=1)
  zeros_da = jnp.zeros((_ZR, _DA), jnp.float32)
  zeros_c = jnp.zeros((_ZR, _C), jnp.float32)

  acc1 = _agg_da(x_aug, src_p, dst_p, zeros_da)
  hs2, hw2, rdeg = _tc1(x, acc1, W_self1, W_neigh1, b1.reshape(1, _H),
                        W_self2, W_neigh2)
  acc2 = _agg_c(hw2, src_p, dst_p, zeros_c)
  return _tc2(hs2, acc2, rdeg, b2.reshape(1, _C))
